# baseline (speedup 1.0000x reference)
import numpy as np
import jax
import jax.numpy as jnp
from jax.experimental import pallas as pl
from jax.experimental.pallas import tpu as pltpu

_NEG = 0.01
_ALPHA = 0.1
_THETA = 0.5
_L = 64
_E = 5
_N = 2048
_WIN = 256
_WOUT = 64
_NC = 32
_NT = 256

_BP = 512
_BQ = 512


def _lrelu(x):
    return jnp.where(x >= 0, x, _NEG * x)


def _prep_kernel(x_ref, w1_ref, b1_ref, w2_ref, xpt_ref):
    h = _lrelu(
        jnp.dot(x_ref[...], w1_ref[...], preferred_element_type=jnp.float32, precision=jax.lax.Precision.HIGHEST)
        + b1_ref[...]
    )
    xpt_ref[...] = _lrelu(
        jax.lax.dot_general(
            w2_ref[...], h, (((0,), (1,)), ((), ())),
            preferred_element_type=jnp.float32, precision=jax.lax.Precision.HIGHEST,
        )
    )


def _sweep_kernel(a_ref, xpt_ref, t_ref, cs_ref, dg_ref, bm_ref, db_ref):
    p = pl.program_id(0)
    q = pl.program_id(1)
    ablk = a_ref[...]
    xpt = xpt_ref[...]

    r = jax.lax.broadcasted_iota(jnp.int32, (_BQ, _BP), 0) + q * _BQ
    c = jax.lax.broadcasted_iota(jnp.int32, (_BQ, _BP), 1) + p * _BP
    dmask = r == c

    accs, css, dgs = [], [], []
    for e in range(_E):
        ae = ablk[e]
        accs.append(
            jax.lax.dot_general(
                xpt, ae, (((1,), (0,)), ((), ())),
                preferred_element_type=jnp.float32, precision=jax.lax.Precision.HIGHEST,
            )
        )
        css.append(jnp.sum(ae, axis=0, keepdims=True))
        dgs.append(jnp.sum(jnp.where(dmask, ae, 0.0), axis=0, keepdims=True))
    t_new = jnp.stack(accs, axis=0)
    cs_new = jnp.stack(css, axis=0)
    dg_new = jnp.stack(dgs, axis=0)

    amean = jnp.sum(ablk, axis=0) * (1.0 / _E)
    bm = jnp.where(dmask, 1.0, (amean != 0.0).astype(jnp.float32))
    bm_ref[...] = bm
    db_new = jnp.sum(bm, axis=0, keepdims=True)

    @pl.when(q == 0)
    def _():
        t_ref[...] = t_new
        cs_ref[...] = cs_new
        dg_ref[...] = dg_new
        db_ref[...] = db_new

    @pl.when(q != 0)
    def _():
        t_ref[...] = t_ref[...] + t_new
        cs_ref[...] = cs_ref[...] + cs_new
        dg_ref[...] = dg_ref[...] + dg_new
        db_ref[...] = db_ref[...] + db_new


def _final_kernel(
    t_ref, cs_ref, dg_ref, xpt_ref, xt_ref, bm_ref, db_ref,
    attw_ref, attb_ref, attq_ref, w0_ref, b0_ref, convp_ref,
    w1g_ref, b1g_ref, l1w_ref, l1b_ref, l2w_ref, l2b_ref,
    tx_ref, tg_ref, nll_ref, y_ref,
):
    xpt = xpt_ref[...]
    attw = attw_ref[...]
    attb_r = attb_ref[...]
    attq_r = attq_ref[...]
    ones_n = jnp.ones((1, _N), jnp.float32)

    def colbc(row):
        return jax.lax.dot_general(
            row, ones_n, (((0,), (0,)), ((), ())),
            preferred_element_type=jnp.float32, precision=jax.lax.Precision.HIGHEST,
        )

    lgs, watts = [], []
    for e in range(_E):
        cs = cs_ref[e]
        dgv = dg_ref[e]
        deg = cs - dgv + 1.0
        inv = jnp.where(deg == 0.0, 0.0, 1.0 / deg)
        lg = jnp.maximum((t_ref[e] + (1.0 - dgv) * xpt) * inv, 0.0)
        lgs.append(lg)
        u = jax.lax.dot_general(
            attw, lg, (((1,), (1,)), ((), ())),
            preferred_element_type=jnp.float32, precision=jax.lax.Precision.HIGHEST,
        )
        watts.append(jnp.sum(attq_r * jnp.tanh(u + attb_r)))

    wmax = watts[0]
    for e in range(1, _E):
        wmax = jnp.maximum(wmax, watts[e])
    exps = [jnp.exp(w - wmax) for w in watts]
    tot = exps[0] + exps[1] + exps[2] + exps[3] + exps[4]
    betas = [ex / tot * float(_E) for ex in exps]

    w0 = w0_ref[...]
    acc = jax.lax.dot_general(
        w0[_E * _WOUT :, :], xt_ref[...], (((0,), (0,)), ((), ())),
        preferred_element_type=jnp.float32, precision=jax.lax.Precision.HIGHEST,
    )
    for e in range(_E):
        acc = acc + betas[e] * jax.lax.dot_general(
            w0[e * _WOUT : (e + 1) * _WOUT, :], lgs[e],
            (((0,), (0,)), ((), ())),
            preferred_element_type=jnp.float32, precision=jax.lax.Precision.HIGHEST,
        )
    x0t = jnp.maximum(acc + colbc(b0_ref[...]), 0.0)

    db = db_ref[...]
    dinv = jnp.where(db > 0.0, jax.lax.rsqrt(db), 0.0)

    bm = bm_ref[...]

    def layer(l, xt):
        ut = xt * dinv
        vt = jax.lax.dot_general(
            ut, bm, (((1,), (0,)), ((), ())),
            preferred_element_type=jnp.float32, precision=jax.lax.Precision.HIGHEST,
        )
        outt = (1.0 - _ALPHA) * (vt * dinv) + _ALPHA * x0t
        wl = convp_ref[l]
        return jnp.maximum(
            jax.lax.dot_general(
                wl, outt, (((0,), (0,)), ((), ())),
                preferred_element_type=jnp.float32, precision=jax.lax.Precision.HIGHEST,
            ),
            0.0,
        )

    xt = jax.lax.fori_loop(0, _L, layer, x0t)

    zt = _lrelu(
        jax.lax.dot_general(
            w1g_ref[...], xt, (((0,), (0,)), ((), ())),
            preferred_element_type=jnp.float32, precision=jax.lax.Precision.HIGHEST,
        )
        + colbc(b1g_ref[...])
    )
    z2t = _lrelu(
        jax.lax.dot_general(
            l1w_ref[...], zt, (((0,), (0,)), ((), ())),
            preferred_element_type=jnp.float32, precision=jax.lax.Precision.HIGHEST,
        )
        + colbc(l1b_ref[...])
    )

    tx = tx_ref[...]
    txb = jax.lax.dot_general(
        tx, ones_n, (((0,), (0,)), ((), ())),
        preferred_element_type=jnp.float32, precision=jax.lax.Precision.HIGHEST,
    )
    onehot = (
        txb
        == jax.lax.broadcasted_iota(jnp.int32, (_NT, _N), 1).astype(jnp.float32)
    ).astype(jnp.float32)
    g = jax.lax.dot_general(
        onehot, z2t, (((1,), (1,)), ((), ())),
        preferred_element_type=jnp.float32, precision=jax.lax.Precision.HIGHEST,
    )
    y = (
        jnp.dot(g, l2w_ref[...], preferred_element_type=jnp.float32, precision=jax.lax.Precision.HIGHEST)
        + l2b_ref[...]
    )
    y_ref[...] = y

    m = jnp.max(y, axis=1, keepdims=True)
    lse = m + jnp.log(jnp.sum(jnp.exp(y - m), axis=1, keepdims=True))
    tg = tg_ref[...]
    ones_nc = jnp.ones((1, _NC), jnp.float32)
    tgb = jax.lax.dot_general(
        tg, ones_nc, (((0,), (0,)), ((), ())),
        preferred_element_type=jnp.float32, precision=jax.lax.Precision.HIGHEST,
    )
    oh_t = (
        tgb
        == jax.lax.broadcasted_iota(jnp.int32, (_NT, _NC), 1).astype(jnp.float32)
    ).astype(jnp.float32)
    picked = jnp.sum(y * oh_t, axis=1, keepdims=True)
    nll_ref[...] = jnp.mean(lse - picked).reshape(1, 1)


def kernel(A, X, target_x, target, weight, attw, attb, attq, linearg_W,
           linearg_b, dg_lin0_W, dg_lin0_b, dg_conv_W, dg_lin1_W, dg_lin1_b,
           l1_W, l1_b, l2_W, l2_b):
    f32 = jnp.float32

    Ap = jnp.transpose(A, (2, 0, 1))
    Xt = X.T
    betas_np = np.log(_THETA / (np.arange(1, _L + 1)) + 1.0).astype(np.float32)
    eye = jnp.eye(_WOUT, dtype=f32)
    convp = (1.0 - betas_np)[:, None, None] * eye + betas_np[:, None, None] * dg_conv_W

    xpt = pl.pallas_call(
        _prep_kernel,
        out_shape=jax.ShapeDtypeStruct((_WOUT, _N), f32),
    )(X, linearg_W, linearg_b.reshape(1, _WIN), weight)

    np_ = _N // _BP
    nq = _N // _BQ
    t_s, cs_s, dg_s, bm_s, db_s = pl.pallas_call(
        _sweep_kernel,
        grid=(np_, nq),
        in_specs=[
            pl.BlockSpec((_E, _BQ, _BP), lambda p, q: (0, q, p)),
            pl.BlockSpec((_WOUT, _BQ), lambda p, q: (0, q)),
        ],
        out_specs=[
            pl.BlockSpec((_E, _WOUT, _BP), lambda p, q: (0, 0, p)),
            pl.BlockSpec((_E, 1, _BP), lambda p, q: (0, 0, p)),
            pl.BlockSpec((_E, 1, _BP), lambda p, q: (0, 0, p)),
            pl.BlockSpec((_BQ, _BP), lambda p, q: (q, p)),
            pl.BlockSpec((1, _BP), lambda p, q: (0, p)),
        ],
        out_shape=[
            jax.ShapeDtypeStruct((_E, _WOUT, _N), f32),
            jax.ShapeDtypeStruct((_E, 1, _N), f32),
            jax.ShapeDtypeStruct((_E, 1, _N), f32),
            jax.ShapeDtypeStruct((_N, _N), f32),
            jax.ShapeDtypeStruct((1, _N), f32),
        ],
        compiler_params=pltpu.CompilerParams(
            dimension_semantics=("arbitrary", "arbitrary"),
        ),
    )(Ap, xpt)

    nll2, y = pl.pallas_call(
        _final_kernel,
        out_shape=[
            jax.ShapeDtypeStruct((1, 1), f32),
            jax.ShapeDtypeStruct((_NT, _NC), f32),
        ],
        compiler_params=pltpu.CompilerParams(
            vmem_limit_bytes=100 * 1024 * 1024,
        ),
    )(
        t_s, cs_s, dg_s, xpt, Xt, bm_s, db_s,
        attw, attb.reshape(1, _WOUT), attq.reshape(1, _WOUT),
        dg_lin0_W, dg_lin0_b.reshape(1, _WOUT), convp,
        dg_lin1_W, dg_lin1_b.reshape(1, _WOUT),
        l1_W, l1_b.reshape(1, _WOUT),
        l2_W, l2_b.reshape(1, _NC),
        target_x.astype(f32).reshape(1, _NT), target.astype(f32).reshape(1, _NT),
    )
    return nll2[0, 0], y


# speedup vs baseline: 2.1865x; 2.1865x over previous
import numpy as np
import jax
import jax.numpy as jnp
from jax.experimental import pallas as pl
from jax.experimental.pallas import tpu as pltpu

_NEG = 0.01
_ALPHA = 0.1
_THETA = 0.5
_L = 64
_E = 5
_N = 2048
_WIN = 256
_WOUT = 64
_NC = 32
_NT = 256

_BP = 512
_BQ = 512


def _lrelu(x):
    return jnp.where(x >= 0, x, _NEG * x)


def _prep_kernel(x_ref, w1_ref, b1_ref, w2_ref, xpt_ref):
    h = _lrelu(
        jnp.dot(x_ref[...], w1_ref[...], preferred_element_type=jnp.float32, precision=jax.lax.Precision.HIGHEST)
        + b1_ref[...]
    )
    xpt_ref[...] = _lrelu(
        jax.lax.dot_general(
            w2_ref[...], h, (((0,), (1,)), ((), ())),
            preferred_element_type=jnp.float32, precision=jax.lax.Precision.HIGHEST,
        )
    )


def _sweep_kernel(a_ref, xpt_ref, t_ref, cs_ref, dg_ref, bm_ref, db_ref):
    p = pl.program_id(0)
    q = pl.program_id(1)
    ablk = a_ref[...]
    xpt = xpt_ref[...]

    r = jax.lax.broadcasted_iota(jnp.int32, (_BQ, _BP), 0) + q * _BQ
    c = jax.lax.broadcasted_iota(jnp.int32, (_BQ, _BP), 1) + p * _BP
    dmask = r == c

    accs, css, dgs = [], [], []
    for e in range(_E):
        ae = ablk[e]
        accs.append(
            jax.lax.dot_general(
                xpt, ae, (((1,), (0,)), ((), ())),
                preferred_element_type=jnp.float32, precision=jax.lax.Precision.HIGHEST,
            )
        )
        css.append(jnp.sum(ae, axis=0, keepdims=True))
        dgs.append(jnp.sum(jnp.where(dmask, ae, 0.0), axis=0, keepdims=True))
    t_new = jnp.stack(accs, axis=0)
    cs_new = jnp.stack(css, axis=0)
    dg_new = jnp.stack(dgs, axis=0)

    amean = jnp.sum(ablk, axis=0) * (1.0 / _E)
    bm = jnp.where(dmask, 1.0, (amean != 0.0).astype(jnp.float32))
    bm_ref[...] = bm.astype(jnp.bfloat16)
    db_new = jnp.sum(bm, axis=0, keepdims=True)

    @pl.when(q == 0)
    def _():
        t_ref[...] = t_new
        cs_ref[...] = cs_new
        dg_ref[...] = dg_new
        db_ref[...] = db_new

    @pl.when(q != 0)
    def _():
        t_ref[...] = t_ref[...] + t_new
        cs_ref[...] = cs_ref[...] + cs_new
        dg_ref[...] = dg_ref[...] + dg_new
        db_ref[...] = db_ref[...] + db_new


def _final_kernel(
    t_ref, cs_ref, dg_ref, xpt_ref, xt_ref, bm_ref, db_ref,
    attw_ref, attb_ref, attq_ref, w0_ref, b0_ref, convp_ref,
    w1g_ref, b1g_ref, l1w_ref, l1b_ref, l2w_ref, l2b_ref,
    tx_ref, tg_ref, nll_ref, y_ref,
):
    xpt = xpt_ref[...]
    attw = attw_ref[...]
    attb_r = attb_ref[...]
    attq_r = attq_ref[...]
    ones_n = jnp.ones((1, _N), jnp.float32)

    def colbc(row):
        return jax.lax.dot_general(
            row, ones_n, (((0,), (0,)), ((), ())),
            preferred_element_type=jnp.float32, precision=jax.lax.Precision.HIGHEST,
        )

    lgs, watts = [], []
    for e in range(_E):
        cs = cs_ref[e]
        dgv = dg_ref[e]
        deg = cs - dgv + 1.0
        inv = jnp.where(deg == 0.0, 0.0, 1.0 / deg)
        lg = jnp.maximum((t_ref[e] + (1.0 - dgv) * xpt) * inv, 0.0)
        lgs.append(lg)
        u = jax.lax.dot_general(
            attw, lg, (((1,), (1,)), ((), ())),
            preferred_element_type=jnp.float32, precision=jax.lax.Precision.HIGHEST,
        )
        watts.append(jnp.sum(attq_r * jnp.tanh(u + attb_r)))

    wmax = watts[0]
    for e in range(1, _E):
        wmax = jnp.maximum(wmax, watts[e])
    exps = [jnp.exp(w - wmax) for w in watts]
    tot = exps[0] + exps[1] + exps[2] + exps[3] + exps[4]
    betas = [ex / tot * float(_E) for ex in exps]

    w0 = w0_ref[...]
    acc = jax.lax.dot_general(
        w0[_E * _WOUT :, :], xt_ref[...], (((0,), (0,)), ((), ())),
        preferred_element_type=jnp.float32, precision=jax.lax.Precision.HIGHEST,
    )
    for e in range(_E):
        acc = acc + betas[e] * jax.lax.dot_general(
            w0[e * _WOUT : (e + 1) * _WOUT, :], lgs[e],
            (((0,), (0,)), ((), ())),
            preferred_element_type=jnp.float32, precision=jax.lax.Precision.HIGHEST,
        )
    x0t = jnp.maximum(acc + colbc(b0_ref[...]), 0.0)

    db = db_ref[...]
    dinv = jnp.where(db > 0.0, jax.lax.rsqrt(db), 0.0)

    bm = bm_ref[...]

    def layer(l, xt):
        ut = xt * dinv
        uh = ut.astype(jnp.bfloat16)
        ul = (ut - uh.astype(jnp.float32)).astype(jnp.bfloat16)
        dn = (((1,), (0,)), ((), ()))
        vt = jax.lax.dot_general(
            uh, bm, dn, preferred_element_type=jnp.float32
        ) + jax.lax.dot_general(
            ul, bm, dn, preferred_element_type=jnp.float32
        )
        outt = (1.0 - _ALPHA) * (vt * dinv) + _ALPHA * x0t
        wl = convp_ref[l]
        return jnp.maximum(
            jax.lax.dot_general(
                wl, outt, (((0,), (0,)), ((), ())),
                preferred_element_type=jnp.float32, precision=jax.lax.Precision.HIGHEST,
            ),
            0.0,
        )

    xt = jax.lax.fori_loop(0, _L, layer, x0t)

    zt = _lrelu(
        jax.lax.dot_general(
            w1g_ref[...], xt, (((0,), (0,)), ((), ())),
            preferred_element_type=jnp.float32, precision=jax.lax.Precision.HIGHEST,
        )
        + colbc(b1g_ref[...])
    )
    z2t = _lrelu(
        jax.lax.dot_general(
            l1w_ref[...], zt, (((0,), (0,)), ((), ())),
            preferred_element_type=jnp.float32, precision=jax.lax.Precision.HIGHEST,
        )
        + colbc(l1b_ref[...])
    )

    tx = tx_ref[...]
    txb = jax.lax.dot_general(
        tx, ones_n, (((0,), (0,)), ((), ())),
        preferred_element_type=jnp.float32, precision=jax.lax.Precision.HIGHEST,
    )
    onehot = (
        txb
        == jax.lax.broadcasted_iota(jnp.int32, (_NT, _N), 1).astype(jnp.float32)
    ).astype(jnp.float32)
    g = jax.lax.dot_general(
        onehot, z2t, (((1,), (1,)), ((), ())),
        preferred_element_type=jnp.float32, precision=jax.lax.Precision.HIGHEST,
    )
    y = (
        jnp.dot(g, l2w_ref[...], preferred_element_type=jnp.float32, precision=jax.lax.Precision.HIGHEST)
        + l2b_ref[...]
    )
    y_ref[...] = y

    m = jnp.max(y, axis=1, keepdims=True)
    lse = m + jnp.log(jnp.sum(jnp.exp(y - m), axis=1, keepdims=True))
    tg = tg_ref[...]
    ones_nc = jnp.ones((1, _NC), jnp.float32)
    tgb = jax.lax.dot_general(
        tg, ones_nc, (((0,), (0,)), ((), ())),
        preferred_element_type=jnp.float32, precision=jax.lax.Precision.HIGHEST,
    )
    oh_t = (
        tgb
        == jax.lax.broadcasted_iota(jnp.int32, (_NT, _NC), 1).astype(jnp.float32)
    ).astype(jnp.float32)
    picked = jnp.sum(y * oh_t, axis=1, keepdims=True)
    nll_ref[...] = jnp.mean(lse - picked).reshape(1, 1)


def kernel(A, X, target_x, target, weight, attw, attb, attq, linearg_W,
           linearg_b, dg_lin0_W, dg_lin0_b, dg_conv_W, dg_lin1_W, dg_lin1_b,
           l1_W, l1_b, l2_W, l2_b):
    f32 = jnp.float32

    Ap = jnp.transpose(A, (2, 0, 1))
    Xt = X.T
    betas_np = np.log(_THETA / (np.arange(1, _L + 1)) + 1.0).astype(np.float32)
    eye = jnp.eye(_WOUT, dtype=f32)
    convp = (1.0 - betas_np)[:, None, None] * eye + betas_np[:, None, None] * dg_conv_W

    xpt = pl.pallas_call(
        _prep_kernel,
        out_shape=jax.ShapeDtypeStruct((_WOUT, _N), f32),
    )(X, linearg_W, linearg_b.reshape(1, _WIN), weight)

    np_ = _N // _BP
    nq = _N // _BQ
    t_s, cs_s, dg_s, bm_s, db_s = pl.pallas_call(
        _sweep_kernel,
        grid=(np_, nq),
        in_specs=[
            pl.BlockSpec((_E, _BQ, _BP), lambda p, q: (0, q, p)),
            pl.BlockSpec((_WOUT, _BQ), lambda p, q: (0, q)),
        ],
        out_specs=[
            pl.BlockSpec((_E, _WOUT, _BP), lambda p, q: (0, 0, p)),
            pl.BlockSpec((_E, 1, _BP), lambda p, q: (0, 0, p)),
            pl.BlockSpec((_E, 1, _BP), lambda p, q: (0, 0, p)),
            pl.BlockSpec((_BQ, _BP), lambda p, q: (q, p)),
            pl.BlockSpec((1, _BP), lambda p, q: (0, p)),
        ],
        out_shape=[
            jax.ShapeDtypeStruct((_E, _WOUT, _N), f32),
            jax.ShapeDtypeStruct((_E, 1, _N), f32),
            jax.ShapeDtypeStruct((_E, 1, _N), f32),
            jax.ShapeDtypeStruct((_N, _N), jnp.bfloat16),
            jax.ShapeDtypeStruct((1, _N), f32),
        ],
        compiler_params=pltpu.CompilerParams(
            dimension_semantics=("arbitrary", "arbitrary"),
        ),
    )(Ap, xpt)

    nll2, y = pl.pallas_call(
        _final_kernel,
        out_shape=[
            jax.ShapeDtypeStruct((1, 1), f32),
            jax.ShapeDtypeStruct((_NT, _NC), f32),
        ],
        compiler_params=pltpu.CompilerParams(
            vmem_limit_bytes=100 * 1024 * 1024,
        ),
    )(
        t_s, cs_s, dg_s, xpt, Xt, bm_s, db_s,
        attw, attb.reshape(1, _WOUT), attq.reshape(1, _WOUT),
        dg_lin0_W, dg_lin0_b.reshape(1, _WOUT), convp,
        dg_lin1_W, dg_lin1_b.reshape(1, _WOUT),
        l1_W, l1_b.reshape(1, _WOUT),
        l2_W, l2_b.reshape(1, _NC),
        target_x.astype(f32).reshape(1, _NT), target.astype(f32).reshape(1, _NT),
    )
    return nll2[0, 0], y


# speedup vs baseline: 3.5485x; 1.6229x over previous
import numpy as np
import jax
import jax.numpy as jnp
from jax.experimental import pallas as pl
from jax.experimental.pallas import tpu as pltpu

_NEG = 0.01
_ALPHA = 0.1
_THETA = 0.5
_L = 64
_E = 5
_N = 2048
_WIN = 256
_WOUT = 64
_NC = 32
_NT = 256

_BP = 512
_BQ = 512


def _lrelu(x):
    return jnp.where(x >= 0, x, _NEG * x)


def _prep_kernel(x_ref, w1_ref, b1_ref, w2_ref, xpt_ref):
    h = _lrelu(
        jnp.dot(x_ref[...], w1_ref[...], preferred_element_type=jnp.float32, precision=jax.lax.Precision.HIGHEST)
        + b1_ref[...]
    )
    xpt_ref[...] = _lrelu(
        jax.lax.dot_general(
            w2_ref[...], h, (((0,), (1,)), ((), ())),
            preferred_element_type=jnp.float32, precision=jax.lax.Precision.HIGHEST,
        )
    )


def _sweep_kernel(a_ref, xpt_ref, t_ref, cs_ref, dg_ref, bm_ref, db_ref):
    p = pl.program_id(0)
    q = pl.program_id(1)
    ablk = a_ref[...]
    xpt = xpt_ref[...]

    r = jax.lax.broadcasted_iota(jnp.int32, (_BQ, _BP), 0) + q * _BQ
    c = jax.lax.broadcasted_iota(jnp.int32, (_BQ, _BP), 1) + p * _BP
    dmask = r == c

    ones_bq = jnp.ones((1, _BQ), jnp.float32)
    accs, css = [], []
    for e in range(_E):
        ae = ablk[e]
        accs.append(
            jax.lax.dot_general(
                xpt, ae, (((1,), (0,)), ((), ())),
                preferred_element_type=jnp.float32, precision=jax.lax.Precision.HIGHEST,
            )
        )
        css.append(
            jax.lax.dot_general(
                ones_bq, ae, (((1,), (0,)), ((), ())),
                preferred_element_type=jnp.float32, precision=jax.lax.Precision.HIGHEST,
            )
        )
    t_new = jnp.stack(accs, axis=0)
    cs_new = jnp.stack(css, axis=0)

    amean = jnp.sum(ablk, axis=0) * (1.0 / _E)
    bmf = jnp.where(dmask, 1.0, (amean != 0.0).astype(jnp.float32))
    bm = bmf.astype(jnp.bfloat16)
    bm_ref[...] = bm
    ones_bq_b = jnp.ones((1, _BQ), jnp.bfloat16)
    db_new = jax.lax.dot_general(
        ones_bq_b, bm, (((1,), (0,)), ((), ())),
        preferred_element_type=jnp.float32,
    )

    @pl.when(q == 0)
    def _():
        t_ref[...] = t_new
        cs_ref[...] = cs_new
        dg_ref[...] = jnp.zeros((_E, 1, _BP), jnp.float32)
        db_ref[...] = db_new

    @pl.when(q != 0)
    def _():
        t_ref[...] = t_ref[...] + t_new
        cs_ref[...] = cs_ref[...] + cs_new
        db_ref[...] = db_ref[...] + db_new

    @pl.when(q == p)
    def _():
        dgs = [
            jnp.sum(jnp.where(dmask, ablk[e], 0.0), axis=0, keepdims=True)
            for e in range(_E)
        ]
        dg_ref[...] = jnp.stack(dgs, axis=0)


def _final_kernel(
    t_ref, cs_ref, dg_ref, xpt_ref, xt_ref, bm_ref, db_ref,
    attw_ref, attb_ref, attq_ref, w0_ref, b0_ref, cvs_ref, cvf_ref,
    w1g_ref, b1g_ref, l1w_ref, l1b_ref, l2w_ref, l2b_ref,
    tx_ref, tg_ref, nll_ref, y_ref,
):
    xpt = xpt_ref[...]
    attw = attw_ref[...]
    attb_r = attb_ref[...]
    attq_r = attq_ref[...]
    ones_n = jnp.ones((1, _N), jnp.float32)

    def colbc(row):
        return jax.lax.dot_general(
            row, ones_n, (((0,), (0,)), ((), ())),
            preferred_element_type=jnp.float32, precision=jax.lax.Precision.HIGHEST,
        )

    lgs, watts = [], []
    for e in range(_E):
        cs = cs_ref[e]
        dgv = dg_ref[e]
        deg = cs - dgv + 1.0
        inv = jnp.where(deg == 0.0, 0.0, 1.0 / deg)
        lg = jnp.maximum((t_ref[e] + (1.0 - dgv) * xpt) * inv, 0.0)
        lgs.append(lg)
        u = jax.lax.dot_general(
            attw, lg, (((1,), (1,)), ((), ())),
            preferred_element_type=jnp.float32, precision=jax.lax.Precision.HIGHEST,
        )
        watts.append(jnp.sum(attq_r * jnp.tanh(u + attb_r)))

    wmax = watts[0]
    for e in range(1, _E):
        wmax = jnp.maximum(wmax, watts[e])
    exps = [jnp.exp(w - wmax) for w in watts]
    tot = exps[0] + exps[1] + exps[2] + exps[3] + exps[4]
    betas = [ex / tot * float(_E) for ex in exps]

    w0 = w0_ref[...]
    acc = jax.lax.dot_general(
        w0[_E * _WOUT :, :], xt_ref[...], (((0,), (0,)), ((), ())),
        preferred_element_type=jnp.float32, precision=jax.lax.Precision.HIGHEST,
    )
    for e in range(_E):
        acc = acc + betas[e] * jax.lax.dot_general(
            w0[e * _WOUT : (e + 1) * _WOUT, :], lgs[e],
            (((0,), (0,)), ((), ())),
            preferred_element_type=jnp.float32, precision=jax.lax.Precision.HIGHEST,
        )
    x0t = jnp.maximum(acc + colbc(b0_ref[...]), 0.0)

    db = db_ref[...]
    dinv = jnp.where(db > 0.0, jax.lax.rsqrt(db), 0.0)

    bm = bm_ref[...]

    def tail(vt, l, xt):
        outt = (1.0 - _ALPHA) * (vt * dinv) + _ALPHA * x0t
        oh = outt.astype(jnp.bfloat16)
        ol = (outt - oh.astype(jnp.float32)).astype(jnp.bfloat16)
        os3 = jnp.concatenate([oh, ol, oh], axis=0)
        ws3 = cvs_ref[l]
        rs = jax.lax.dot_general(
            ws3, os3, (((0,), (0,)), ((), ())),
            preferred_element_type=jnp.float32,
        )
        return jnp.maximum(rs, 0.0)

    def layer_generic(l, xt):
        ut = xt * dinv
        uh = ut.astype(jnp.bfloat16)
        ul = (ut - uh.astype(jnp.float32)).astype(jnp.bfloat16)
        us = jnp.concatenate([uh, ul], axis=0)
        vs = jax.lax.dot_general(
            us, bm, (((1,), (0,)), ((), ())),
            preferred_element_type=jnp.float32,
        )
        vt = vs[:_WOUT, :] + vs[_WOUT:, :]
        return tail(vt, l, xt)

    def rank1_loop(x0):
        xt = x0
        for l in range(_L):
            s = jax.lax.dot_general(
                xt, dinv, (((1,), (1,)), ((), ())),
                preferred_element_type=jnp.float32, precision=jax.lax.Precision.HIGHEST,
            )
            od = jax.lax.dot_general(
                s, dinv, (((1,), (0,)), ((), ())),
                preferred_element_type=jnp.float32, precision=jax.lax.Precision.HIGHEST,
            )
            outt = (1.0 - _ALPHA) * od + _ALPHA * x0t
            xt = jnp.maximum(
                jax.lax.dot_general(
                    cvf_ref[l], outt, (((0,), (0,)), ((), ())),
                    preferred_element_type=jnp.float32,
                    precision=jax.lax.Precision.HIGHEST,
                ),
                0.0,
            )
        return xt

    allones = jnp.sum(db) == float(_N) * float(_N)
    xt = jax.lax.cond(
        allones,
        rank1_loop,
        lambda x0: jax.lax.fori_loop(0, _L, layer_generic, x0),
        x0t,
    )

    zt = _lrelu(
        jax.lax.dot_general(
            w1g_ref[...], xt, (((0,), (0,)), ((), ())),
            preferred_element_type=jnp.float32, precision=jax.lax.Precision.HIGHEST,
        )
        + colbc(b1g_ref[...])
    )
    z2t = _lrelu(
        jax.lax.dot_general(
            l1w_ref[...], zt, (((0,), (0,)), ((), ())),
            preferred_element_type=jnp.float32, precision=jax.lax.Precision.HIGHEST,
        )
        + colbc(l1b_ref[...])
    )

    tx = tx_ref[...]
    txb = jax.lax.dot_general(
        tx, ones_n, (((0,), (0,)), ((), ())),
        preferred_element_type=jnp.float32, precision=jax.lax.Precision.HIGHEST,
    )
    onehot = (
        txb
        == jax.lax.broadcasted_iota(jnp.int32, (_NT, _N), 1).astype(jnp.float32)
    ).astype(jnp.float32)
    g = jax.lax.dot_general(
        onehot, z2t, (((1,), (1,)), ((), ())),
        preferred_element_type=jnp.float32, precision=jax.lax.Precision.HIGHEST,
    )
    y = (
        jnp.dot(g, l2w_ref[...], preferred_element_type=jnp.float32, precision=jax.lax.Precision.HIGHEST)
        + l2b_ref[...]
    )
    y_ref[...] = y

    m = jnp.max(y, axis=1, keepdims=True)
    lse = m + jnp.log(jnp.sum(jnp.exp(y - m), axis=1, keepdims=True))
    tg = tg_ref[...]
    ones_nc = jnp.ones((1, _NC), jnp.float32)
    tgb = jax.lax.dot_general(
        tg, ones_nc, (((0,), (0,)), ((), ())),
        preferred_element_type=jnp.float32, precision=jax.lax.Precision.HIGHEST,
    )
    oh_t = (
        tgb
        == jax.lax.broadcasted_iota(jnp.int32, (_NT, _NC), 1).astype(jnp.float32)
    ).astype(jnp.float32)
    picked = jnp.sum(y * oh_t, axis=1, keepdims=True)
    nll_ref[...] = jnp.mean(lse - picked).reshape(1, 1)


def kernel(A, X, target_x, target, weight, attw, attb, attq, linearg_W,
           linearg_b, dg_lin0_W, dg_lin0_b, dg_conv_W, dg_lin1_W, dg_lin1_b,
           l1_W, l1_b, l2_W, l2_b):
    f32 = jnp.float32

    Ap = jnp.transpose(A, (2, 0, 1))
    Xt = X.T
    betas_np = np.log(_THETA / (np.arange(1, _L + 1)) + 1.0).astype(np.float32)
    eye = jnp.eye(_WOUT, dtype=f32)
    convp = (1.0 - betas_np)[:, None, None] * eye + betas_np[:, None, None] * dg_conv_W
    cv_hi = convp.astype(jnp.bfloat16)
    cv_lo = (convp - cv_hi.astype(f32)).astype(jnp.bfloat16)
    cvs = jnp.concatenate([cv_hi, cv_hi, cv_lo], axis=1)

    xpt = pl.pallas_call(
        _prep_kernel,
        out_shape=jax.ShapeDtypeStruct((_WOUT, _N), f32),
    )(X, linearg_W, linearg_b.reshape(1, _WIN), weight)

    np_ = _N // _BP
    nq = _N // _BQ
    t_s, cs_s, dg_s, bm_s, db_s = pl.pallas_call(
        _sweep_kernel,
        grid=(np_, nq),
        in_specs=[
            pl.BlockSpec((_E, _BQ, _BP), lambda p, q: (0, q, p)),
            pl.BlockSpec((_WOUT, _BQ), lambda p, q: (0, q)),
        ],
        out_specs=[
            pl.BlockSpec((_E, _WOUT, _BP), lambda p, q: (0, 0, p)),
            pl.BlockSpec((_E, 1, _BP), lambda p, q: (0, 0, p)),
            pl.BlockSpec((_E, 1, _BP), lambda p, q: (0, 0, p)),
            pl.BlockSpec((_BQ, _BP), lambda p, q: (q, p)),
            pl.BlockSpec((1, _BP), lambda p, q: (0, p)),
        ],
        out_shape=[
            jax.ShapeDtypeStruct((_E, _WOUT, _N), f32),
            jax.ShapeDtypeStruct((_E, 1, _N), f32),
            jax.ShapeDtypeStruct((_E, 1, _N), f32),
            jax.ShapeDtypeStruct((_N, _N), jnp.bfloat16),
            jax.ShapeDtypeStruct((1, _N), f32),
        ],
        compiler_params=pltpu.CompilerParams(
            dimension_semantics=("arbitrary", "arbitrary"),
        ),
    )(Ap, xpt)

    nll2, y = pl.pallas_call(
        _final_kernel,
        out_shape=[
            jax.ShapeDtypeStruct((1, 1), f32),
            jax.ShapeDtypeStruct((_NT, _NC), f32),
        ],
        compiler_params=pltpu.CompilerParams(
            vmem_limit_bytes=100 * 1024 * 1024,
        ),
    )(
        t_s, cs_s, dg_s, xpt, Xt, bm_s, db_s,
        attw, attb.reshape(1, _WOUT), attq.reshape(1, _WOUT),
        dg_lin0_W, dg_lin0_b.reshape(1, _WOUT), cvs, convp,
        dg_lin1_W, dg_lin1_b.reshape(1, _WOUT),
        l1_W, l1_b.reshape(1, _WOUT),
        l2_W, l2_b.reshape(1, _NC),
        target_x.astype(f32).reshape(1, _NT), target.astype(f32).reshape(1, _NT),
    )
    return nll2[0, 0], y


# speedup vs baseline: 4.4374x; 1.2505x over previous
import numpy as np
import jax
import jax.numpy as jnp
from jax.experimental import pallas as pl
from jax.experimental.pallas import tpu as pltpu

_NEG = 0.01
_ALPHA = 0.1
_THETA = 0.5
_L = 64
_E = 5
_N = 2048
_WIN = 256
_WOUT = 64
_NC = 32
_NT = 256

_BP = 512
_BQ = 512


def _lrelu(x):
    return jnp.where(x >= 0, x, _NEG * x)


def _prep_kernel(x_ref, w1_ref, b1_ref, w2_ref, xpt_ref):
    h = _lrelu(
        jnp.dot(x_ref[...], w1_ref[...], preferred_element_type=jnp.float32, precision=jax.lax.Precision.HIGHEST)
        + b1_ref[...]
    )
    xpt_ref[...] = _lrelu(
        jax.lax.dot_general(
            w2_ref[...], h, (((0,), (1,)), ((), ())),
            preferred_element_type=jnp.float32, precision=jax.lax.Precision.HIGHEST,
        )
    )


def _sweep_kernel(a_ref, xpt_ref, t_ref, cs_ref, dg_ref, bm_ref, db_ref):
    p = pl.program_id(0)
    q = pl.program_id(1)
    ablk = a_ref[...]
    xpt = xpt_ref[...]

    r = jax.lax.broadcasted_iota(jnp.int32, (_BQ, _BP), 0) + q * _BQ
    c = jax.lax.broadcasted_iota(jnp.int32, (_BQ, _BP), 1) + p * _BP
    dmask = r == c

    accs, css, dgs = [], [], []
    for e in range(_E):
        ae = ablk[e]
        accs.append(
            jax.lax.dot_general(
                xpt, ae, (((1,), (0,)), ((), ())),
                preferred_element_type=jnp.float32, precision=jax.lax.Precision.HIGHEST,
            )
        )
        css.append(jnp.sum(ae, axis=0, keepdims=True))
        dgs.append(jnp.sum(jnp.where(dmask, ae, 0.0), axis=0, keepdims=True))
    t_new = jnp.stack(accs, axis=0)
    cs_new = jnp.stack(css, axis=0)
    dg_new = jnp.stack(dgs, axis=0)

    amean = jnp.sum(ablk, axis=0) * (1.0 / _E)
    bm = jnp.where(dmask, 1.0, (amean != 0.0).astype(jnp.float32))
    bm_ref[...] = bm.astype(jnp.bfloat16)
    db_new = jnp.sum(bm, axis=0, keepdims=True)

    @pl.when(q == 0)
    def _():
        t_ref[...] = t_new
        cs_ref[...] = cs_new
        dg_ref[...] = dg_new
        db_ref[...] = db_new

    @pl.when(q != 0)
    def _():
        t_ref[...] = t_ref[...] + t_new
        cs_ref[...] = cs_ref[...] + cs_new
        dg_ref[...] = dg_ref[...] + dg_new
        db_ref[...] = db_ref[...] + db_new


def _final_kernel(
    t_ref, cs_ref, dg_ref, xpt_ref, xt_ref, bm_ref, db_ref,
    attw_ref, attb_ref, attq_ref, w0_ref, b0_ref, cvs_ref,
    w1g_ref, b1g_ref, l1w_ref, l1b_ref, l2w_ref, l2b_ref,
    tx_ref, tg_ref, nll_ref, y_ref,
):
    xpt = xpt_ref[...]
    attw = attw_ref[...]
    attb_r = attb_ref[...]
    attq_r = attq_ref[...]
    ones_n = jnp.ones((1, _N), jnp.float32)

    def colbc(row):
        return jax.lax.dot_general(
            row, ones_n, (((0,), (0,)), ((), ())),
            preferred_element_type=jnp.float32, precision=jax.lax.Precision.HIGHEST,
        )

    lgs, watts = [], []
    for e in range(_E):
        cs = cs_ref[e]
        dgv = dg_ref[e]
        deg = cs - dgv + 1.0
        inv = jnp.where(deg == 0.0, 0.0, 1.0 / deg)
        lg = jnp.maximum((t_ref[e] + (1.0 - dgv) * xpt) * inv, 0.0)
        lgs.append(lg)
        u = jax.lax.dot_general(
            attw, lg, (((1,), (1,)), ((), ())),
            preferred_element_type=jnp.float32, precision=jax.lax.Precision.HIGHEST,
        )
        watts.append(jnp.sum(attq_r * jnp.tanh(u + attb_r)))

    wmax = watts[0]
    for e in range(1, _E):
        wmax = jnp.maximum(wmax, watts[e])
    exps = [jnp.exp(w - wmax) for w in watts]
    tot = exps[0] + exps[1] + exps[2] + exps[3] + exps[4]
    betas = [ex / tot * float(_E) for ex in exps]

    w0 = w0_ref[...]
    acc = jax.lax.dot_general(
        w0[_E * _WOUT :, :], xt_ref[...], (((0,), (0,)), ((), ())),
        preferred_element_type=jnp.float32, precision=jax.lax.Precision.HIGHEST,
    )
    for e in range(_E):
        acc = acc + betas[e] * jax.lax.dot_general(
            w0[e * _WOUT : (e + 1) * _WOUT, :], lgs[e],
            (((0,), (0,)), ((), ())),
            preferred_element_type=jnp.float32, precision=jax.lax.Precision.HIGHEST,
        )
    x0t = jnp.maximum(acc + colbc(b0_ref[...]), 0.0)

    db = db_ref[...]
    dinv = jnp.where(db > 0.0, jax.lax.rsqrt(db), 0.0)

    bm = bm_ref[...]

    def tail(vt, l, xt):
        outt = (1.0 - _ALPHA) * (vt * dinv) + _ALPHA * x0t
        oh = outt.astype(jnp.bfloat16)
        ol = (outt - oh.astype(jnp.float32)).astype(jnp.bfloat16)
        os3 = jnp.concatenate([oh, ol, oh], axis=0)
        ws3 = cvs_ref[l]
        rs = jax.lax.dot_general(
            ws3, os3, (((0,), (0,)), ((), ())),
            preferred_element_type=jnp.float32,
        )
        return jnp.maximum(rs, 0.0)

    def layer_generic(l, xt):
        ut = xt * dinv
        uh = ut.astype(jnp.bfloat16)
        ul = (ut - uh.astype(jnp.float32)).astype(jnp.bfloat16)
        us = jnp.concatenate([uh, ul], axis=0)
        vs = jax.lax.dot_general(
            us, bm, (((1,), (0,)), ((), ())),
            preferred_element_type=jnp.float32,
        )
        vt = vs[:_WOUT, :] + vs[_WOUT:, :]
        return tail(vt, l, xt)

    def layer_rank1(l, xt):
        s = jax.lax.dot_general(
            xt, dinv, (((1,), (1,)), ((), ())),
            preferred_element_type=jnp.float32, precision=jax.lax.Precision.HIGHEST,
        )
        od = jax.lax.dot_general(
            s, dinv, (((1,), (0,)), ((), ())),
            preferred_element_type=jnp.float32, precision=jax.lax.Precision.HIGHEST,
        )
        outt = (1.0 - _ALPHA) * od + _ALPHA * x0t
        oh = outt.astype(jnp.bfloat16)
        ol = (outt - oh.astype(jnp.float32)).astype(jnp.bfloat16)
        os3 = jnp.concatenate([oh, ol, oh], axis=0)
        ws3 = cvs_ref[l]
        rs = jax.lax.dot_general(
            ws3, os3, (((0,), (0,)), ((), ())),
            preferred_element_type=jnp.float32,
        )
        return jnp.maximum(rs, 0.0)

    allones = jnp.sum(db) == float(_N) * float(_N)
    xt = jax.lax.cond(
        allones,
        lambda x0: jax.lax.fori_loop(0, _L, layer_rank1, x0),
        lambda x0: jax.lax.fori_loop(0, _L, layer_generic, x0),
        x0t,
    )

    zt = _lrelu(
        jax.lax.dot_general(
            w1g_ref[...], xt, (((0,), (0,)), ((), ())),
            preferred_element_type=jnp.float32, precision=jax.lax.Precision.HIGHEST,
        )
        + colbc(b1g_ref[...])
    )
    z2t = _lrelu(
        jax.lax.dot_general(
            l1w_ref[...], zt, (((0,), (0,)), ((), ())),
            preferred_element_type=jnp.float32, precision=jax.lax.Precision.HIGHEST,
        )
        + colbc(l1b_ref[...])
    )

    tx = tx_ref[...]
    txb = jax.lax.dot_general(
        tx, ones_n, (((0,), (0,)), ((), ())),
        preferred_element_type=jnp.float32, precision=jax.lax.Precision.HIGHEST,
    )
    onehot = (
        txb
        == jax.lax.broadcasted_iota(jnp.int32, (_NT, _N), 1).astype(jnp.float32)
    ).astype(jnp.float32)
    g = jax.lax.dot_general(
        onehot, z2t, (((1,), (1,)), ((), ())),
        preferred_element_type=jnp.float32, precision=jax.lax.Precision.HIGHEST,
    )
    y = (
        jnp.dot(g, l2w_ref[...], preferred_element_type=jnp.float32, precision=jax.lax.Precision.HIGHEST)
        + l2b_ref[...]
    )
    y_ref[...] = y

    m = jnp.max(y, axis=1, keepdims=True)
    lse = m + jnp.log(jnp.sum(jnp.exp(y - m), axis=1, keepdims=True))
    tg = tg_ref[...]
    ones_nc = jnp.ones((1, _NC), jnp.float32)
    tgb = jax.lax.dot_general(
        tg, ones_nc, (((0,), (0,)), ((), ())),
        preferred_element_type=jnp.float32, precision=jax.lax.Precision.HIGHEST,
    )
    oh_t = (
        tgb
        == jax.lax.broadcasted_iota(jnp.int32, (_NT, _NC), 1).astype(jnp.float32)
    ).astype(jnp.float32)
    picked = jnp.sum(y * oh_t, axis=1, keepdims=True)
    nll_ref[...] = jnp.mean(lse - picked).reshape(1, 1)


def kernel(A, X, target_x, target, weight, attw, attb, attq, linearg_W,
           linearg_b, dg_lin0_W, dg_lin0_b, dg_conv_W, dg_lin1_W, dg_lin1_b,
           l1_W, l1_b, l2_W, l2_b):
    f32 = jnp.float32

    Ap = jnp.transpose(A, (2, 0, 1))
    Xt = X.T
    betas_np = np.log(_THETA / (np.arange(1, _L + 1)) + 1.0).astype(np.float32)
    eye = jnp.eye(_WOUT, dtype=f32)
    convp = (1.0 - betas_np)[:, None, None] * eye + betas_np[:, None, None] * dg_conv_W
    cv_hi = convp.astype(jnp.bfloat16)
    cv_lo = (convp - cv_hi.astype(f32)).astype(jnp.bfloat16)
    cvs = jnp.concatenate([cv_hi, cv_hi, cv_lo], axis=1)

    xpt = pl.pallas_call(
        _prep_kernel,
        out_shape=jax.ShapeDtypeStruct((_WOUT, _N), f32),
    )(X, linearg_W, linearg_b.reshape(1, _WIN), weight)

    np_ = _N // _BP
    nq = _N // _BQ
    t_s, cs_s, dg_s, bm_s, db_s = pl.pallas_call(
        _sweep_kernel,
        grid=(np_, nq),
        in_specs=[
            pl.BlockSpec((_E, _BQ, _BP), lambda p, q: (0, q, p)),
            pl.BlockSpec((_WOUT, _BQ), lambda p, q: (0, q)),
        ],
        out_specs=[
            pl.BlockSpec((_E, _WOUT, _BP), lambda p, q: (0, 0, p)),
            pl.BlockSpec((_E, 1, _BP), lambda p, q: (0, 0, p)),
            pl.BlockSpec((_E, 1, _BP), lambda p, q: (0, 0, p)),
            pl.BlockSpec((_BQ, _BP), lambda p, q: (q, p)),
            pl.BlockSpec((1, _BP), lambda p, q: (0, p)),
        ],
        out_shape=[
            jax.ShapeDtypeStruct((_E, _WOUT, _N), f32),
            jax.ShapeDtypeStruct((_E, 1, _N), f32),
            jax.ShapeDtypeStruct((_E, 1, _N), f32),
            jax.ShapeDtypeStruct((_N, _N), jnp.bfloat16),
            jax.ShapeDtypeStruct((1, _N), f32),
        ],
        compiler_params=pltpu.CompilerParams(
            dimension_semantics=("arbitrary", "arbitrary"),
        ),
    )(Ap, xpt)

    nll2, y = pl.pallas_call(
        _final_kernel,
        out_shape=[
            jax.ShapeDtypeStruct((1, 1), f32),
            jax.ShapeDtypeStruct((_NT, _NC), f32),
        ],
        compiler_params=pltpu.CompilerParams(
            vmem_limit_bytes=100 * 1024 * 1024,
        ),
    )(
        t_s, cs_s, dg_s, xpt, Xt, bm_s, db_s,
        attw, attb.reshape(1, _WOUT), attq.reshape(1, _WOUT),
        dg_lin0_W, dg_lin0_b.reshape(1, _WOUT), cvs,
        dg_lin1_W, dg_lin1_b.reshape(1, _WOUT),
        l1_W, l1_b.reshape(1, _WOUT),
        l2_W, l2_b.reshape(1, _NC),
        target_x.astype(f32).reshape(1, _NT), target.astype(f32).reshape(1, _NT),
    )
    return nll2[0, 0], y


# speedup vs baseline: 4.9196x; 1.1087x over previous
import numpy as np
import jax
import jax.numpy as jnp
from jax.experimental import pallas as pl
from jax.experimental.pallas import tpu as pltpu

_NEG = 0.01
_ALPHA = 0.1
_THETA = 0.5
_L = 64
_E = 5
_N = 2048
_WIN = 256
_WOUT = 64
_NC = 32
_NT = 256

_BP = 512
_BQ = 512


def _lrelu(x):
    return jnp.where(x >= 0, x, _NEG * x)


def _prep_kernel(x_ref, w1_ref, b1_ref, w2_ref, xpt_ref):
    h = _lrelu(
        jnp.dot(x_ref[...], w1_ref[...], preferred_element_type=jnp.float32, precision=jax.lax.Precision.HIGHEST)
        + b1_ref[...]
    )
    xpt_ref[...] = _lrelu(
        jax.lax.dot_general(
            w2_ref[...], h, (((0,), (1,)), ((), ())),
            preferred_element_type=jnp.float32, precision=jax.lax.Precision.HIGHEST,
        )
    )


def _sweep_kernel(a_ref, xpt_ref, t_ref, cs_ref, dg_ref, bm_ref, db_ref):
    p = pl.program_id(0)
    q = pl.program_id(1)
    ablk = a_ref[...]
    xpt = xpt_ref[...]

    r = jax.lax.broadcasted_iota(jnp.int32, (_BQ, _BP), 0) + q * _BQ
    c = jax.lax.broadcasted_iota(jnp.int32, (_BQ, _BP), 1) + p * _BP
    dmask = r == c

    accs, css = [], []
    for e in range(_E):
        ae = ablk[e]
        accs.append(
            jax.lax.dot_general(
                xpt, ae, (((1,), (0,)), ((), ())),
                preferred_element_type=jnp.float32, precision=jax.lax.Precision.HIGHEST,
            )
        )
        css.append(jnp.sum(ae, axis=0, keepdims=True))
    t_new = jnp.stack(accs, axis=0)
    cs_new = jnp.stack(css, axis=0)

    amean = jnp.sum(ablk, axis=0) * (1.0 / _E)
    bm = jnp.where(dmask, 1.0, (amean != 0.0).astype(jnp.float32))
    bm_ref[...] = bm.astype(jnp.bfloat16)
    db_new = jnp.sum(bm, axis=0, keepdims=True)

    @pl.when(q == 0)
    def _():
        t_ref[...] = t_new
        cs_ref[...] = cs_new
        dg_ref[...] = jnp.zeros((_E, 1, _BP), jnp.float32)
        db_ref[...] = db_new

    @pl.when(q != 0)
    def _():
        t_ref[...] = t_ref[...] + t_new
        cs_ref[...] = cs_ref[...] + cs_new
        db_ref[...] = db_ref[...] + db_new

    @pl.when(q == p)
    def _():
        dgs = [
            jnp.sum(jnp.where(dmask, ablk[e], 0.0), axis=0, keepdims=True)
            for e in range(_E)
        ]
        dg_ref[...] = jnp.stack(dgs, axis=0)


def _final_kernel(
    t_ref, cs_ref, dg_ref, xpt_ref, xt_ref, bm_ref, db_ref,
    attw_ref, attb_ref, attq_ref, w0_ref, b0_ref, cvs_ref,
    w1g_ref, b1g_ref, l1w_ref, l1b_ref, l2w_ref, l2b_ref,
    tx_ref, tg_ref, nll_ref, y_ref,
):
    xpt = xpt_ref[...]
    attw = attw_ref[...]
    attb_r = attb_ref[...]
    attq_r = attq_ref[...]
    ones_n = jnp.ones((1, _N), jnp.float32)

    def colbc(row):
        return jax.lax.dot_general(
            row, ones_n, (((0,), (0,)), ((), ())),
            preferred_element_type=jnp.float32, precision=jax.lax.Precision.HIGHEST,
        )

    lgs, watts = [], []
    for e in range(_E):
        cs = cs_ref[e]
        dgv = dg_ref[e]
        deg = cs - dgv + 1.0
        inv = jnp.where(deg == 0.0, 0.0, 1.0 / deg)
        lg = jnp.maximum((t_ref[e] + (1.0 - dgv) * xpt) * inv, 0.0)
        lgs.append(lg)
        u = jax.lax.dot_general(
            attw, lg, (((1,), (1,)), ((), ())),
            preferred_element_type=jnp.float32, precision=jax.lax.Precision.HIGHEST,
        )
        watts.append(jnp.sum(attq_r * jnp.tanh(u + attb_r)))

    wmax = watts[0]
    for e in range(1, _E):
        wmax = jnp.maximum(wmax, watts[e])
    exps = [jnp.exp(w - wmax) for w in watts]
    tot = exps[0] + exps[1] + exps[2] + exps[3] + exps[4]
    betas = [ex / tot * float(_E) for ex in exps]

    w0 = w0_ref[...]
    acc = jax.lax.dot_general(
        w0[_E * _WOUT :, :], xt_ref[...], (((0,), (0,)), ((), ())),
        preferred_element_type=jnp.float32, precision=jax.lax.Precision.HIGHEST,
    )
    for e in range(_E):
        acc = acc + betas[e] * jax.lax.dot_general(
            w0[e * _WOUT : (e + 1) * _WOUT, :], lgs[e],
            (((0,), (0,)), ((), ())),
            preferred_element_type=jnp.float32, precision=jax.lax.Precision.HIGHEST,
        )
    x0t = jnp.maximum(acc + colbc(b0_ref[...]), 0.0)

    db = db_ref[...]
    dinv = jnp.where(db > 0.0, jax.lax.rsqrt(db), 0.0)

    bm = bm_ref[...]

    def tail(vt, l, xt):
        outt = (1.0 - _ALPHA) * (vt * dinv) + _ALPHA * x0t
        oh = outt.astype(jnp.bfloat16)
        ol = (outt - oh.astype(jnp.float32)).astype(jnp.bfloat16)
        os3 = jnp.concatenate([oh, ol, oh], axis=0)
        ws3 = cvs_ref[l]
        rs = jax.lax.dot_general(
            ws3, os3, (((0,), (0,)), ((), ())),
            preferred_element_type=jnp.float32,
        )
        return jnp.maximum(rs, 0.0)

    def layer_generic(l, xt):
        ut = xt * dinv
        uh = ut.astype(jnp.bfloat16)
        ul = (ut - uh.astype(jnp.float32)).astype(jnp.bfloat16)
        us = jnp.concatenate([uh, ul], axis=0)
        vs = jax.lax.dot_general(
            us, bm, (((1,), (0,)), ((), ())),
            preferred_element_type=jnp.float32,
        )
        vt = vs[:_WOUT, :] + vs[_WOUT:, :]
        return tail(vt, l, xt)

    def layer_rank1(l, xt):
        ut = xt * dinv
        s = jax.lax.dot_general(
            ut, ones_n, (((1,), (1,)), ((), ())),
            preferred_element_type=jnp.float32, precision=jax.lax.Precision.HIGHEST,
        )
        vt = jax.lax.dot_general(
            s, ones_n, (((1,), (0,)), ((), ())),
            preferred_element_type=jnp.float32, precision=jax.lax.Precision.HIGHEST,
        )
        return tail(vt, l, xt)

    allones = jnp.sum(db) == float(_N) * float(_N)
    xt = jax.lax.cond(
        allones,
        lambda x0: jax.lax.fori_loop(0, _L, layer_rank1, x0),
        lambda x0: jax.lax.fori_loop(0, _L, layer_generic, x0),
        x0t,
    )

    zt = _lrelu(
        jax.lax.dot_general(
            w1g_ref[...], xt, (((0,), (0,)), ((), ())),
            preferred_element_type=jnp.float32, precision=jax.lax.Precision.HIGHEST,
        )
        + colbc(b1g_ref[...])
    )
    z2t = _lrelu(
        jax.lax.dot_general(
            l1w_ref[...], zt, (((0,), (0,)), ((), ())),
            preferred_element_type=jnp.float32, precision=jax.lax.Precision.HIGHEST,
        )
        + colbc(l1b_ref[...])
    )

    tx = tx_ref[...]
    txb = jax.lax.dot_general(
        tx, ones_n, (((0,), (0,)), ((), ())),
        preferred_element_type=jnp.float32, precision=jax.lax.Precision.HIGHEST,
    )
    onehot = (
        txb
        == jax.lax.broadcasted_iota(jnp.int32, (_NT, _N), 1).astype(jnp.float32)
    ).astype(jnp.float32)
    g = jax.lax.dot_general(
        onehot, z2t, (((1,), (1,)), ((), ())),
        preferred_element_type=jnp.float32, precision=jax.lax.Precision.HIGHEST,
    )
    y = (
        jnp.dot(g, l2w_ref[...], preferred_element_type=jnp.float32, precision=jax.lax.Precision.HIGHEST)
        + l2b_ref[...]
    )
    y_ref[...] = y

    m = jnp.max(y, axis=1, keepdims=True)
    lse = m + jnp.log(jnp.sum(jnp.exp(y - m), axis=1, keepdims=True))
    tg = tg_ref[...]
    ones_nc = jnp.ones((1, _NC), jnp.float32)
    tgb = jax.lax.dot_general(
        tg, ones_nc, (((0,), (0,)), ((), ())),
        preferred_element_type=jnp.float32, precision=jax.lax.Precision.HIGHEST,
    )
    oh_t = (
        tgb
        == jax.lax.broadcasted_iota(jnp.int32, (_NT, _NC), 1).astype(jnp.float32)
    ).astype(jnp.float32)
    picked = jnp.sum(y * oh_t, axis=1, keepdims=True)
    nll_ref[...] = jnp.mean(lse - picked).reshape(1, 1)


def kernel(A, X, target_x, target, weight, attw, attb, attq, linearg_W,
           linearg_b, dg_lin0_W, dg_lin0_b, dg_conv_W, dg_lin1_W, dg_lin1_b,
           l1_W, l1_b, l2_W, l2_b):
    f32 = jnp.float32

    Ap = jnp.transpose(A, (2, 0, 1))
    Xt = X.T
    betas_np = np.log(_THETA / (np.arange(1, _L + 1)) + 1.0).astype(np.float32)
    eye = jnp.eye(_WOUT, dtype=f32)
    convp = (1.0 - betas_np)[:, None, None] * eye + betas_np[:, None, None] * dg_conv_W
    cv_hi = convp.astype(jnp.bfloat16)
    cv_lo = (convp - cv_hi.astype(f32)).astype(jnp.bfloat16)
    cvs = jnp.concatenate([cv_hi, cv_hi, cv_lo], axis=1)

    xpt = pl.pallas_call(
        _prep_kernel,
        out_shape=jax.ShapeDtypeStruct((_WOUT, _N), f32),
    )(X, linearg_W, linearg_b.reshape(1, _WIN), weight)

    np_ = _N // _BP
    nq = _N // _BQ
    t_s, cs_s, dg_s, bm_s, db_s = pl.pallas_call(
        _sweep_kernel,
        grid=(np_, nq),
        in_specs=[
            pl.BlockSpec((_E, _BQ, _BP), lambda p, q: (0, q, p)),
            pl.BlockSpec((_WOUT, _BQ), lambda p, q: (0, q)),
        ],
        out_specs=[
            pl.BlockSpec((_E, _WOUT, _BP), lambda p, q: (0, 0, p)),
            pl.BlockSpec((_E, 1, _BP), lambda p, q: (0, 0, p)),
            pl.BlockSpec((_E, 1, _BP), lambda p, q: (0, 0, p)),
            pl.BlockSpec((_BQ, _BP), lambda p, q: (q, p)),
            pl.BlockSpec((1, _BP), lambda p, q: (0, p)),
        ],
        out_shape=[
            jax.ShapeDtypeStruct((_E, _WOUT, _N), f32),
            jax.ShapeDtypeStruct((_E, 1, _N), f32),
            jax.ShapeDtypeStruct((_E, 1, _N), f32),
            jax.ShapeDtypeStruct((_N, _N), jnp.bfloat16),
            jax.ShapeDtypeStruct((1, _N), f32),
        ],
        compiler_params=pltpu.CompilerParams(
            dimension_semantics=("arbitrary", "arbitrary"),
        ),
    )(Ap, xpt)

    nll2, y = pl.pallas_call(
        _final_kernel,
        out_shape=[
            jax.ShapeDtypeStruct((1, 1), f32),
            jax.ShapeDtypeStruct((_NT, _NC), f32),
        ],
        compiler_params=pltpu.CompilerParams(
            vmem_limit_bytes=100 * 1024 * 1024,
        ),
    )(
        t_s, cs_s, dg_s, xpt, Xt, bm_s, db_s,
        attw, attb.reshape(1, _WOUT), attq.reshape(1, _WOUT),
        dg_lin0_W, dg_lin0_b.reshape(1, _WOUT), cvs,
        dg_lin1_W, dg_lin1_b.reshape(1, _WOUT),
        l1_W, l1_b.reshape(1, _WOUT),
        l2_W, l2_b.reshape(1, _NC),
        target_x.astype(f32).reshape(1, _NT), target.astype(f32).reshape(1, _NT),
    )
    return nll2[0, 0], y
